# baseline (speedup 1.0000x reference)
import functools

import jax
import jax.numpy as jnp
from jax import lax
from jax.experimental import pallas as pl
from jax.experimental.pallas import tpu as pltpu
from jax.experimental.pallas import tpu_sc as plsc

MAX_OBJECTS = 16384
SLOTS = 8
D_MEMORY = 256
M = 4096

NC = 2
NS = 16
NW = NC * NS
BPW = M // NW
CH = 8
NCH = BPW // CH
KHALF = MAX_OBJECTS // 2
NGRP = M // 16
UNROLL = 4


def _gather_body(idx_hbm, mem_hbm, pose_hbm, mask_hbm,
                 mem_out, pose_out, mask_out,
                 idx_v, mem_v, pose_v, pout_v, mask_tab_v, mask_loc_v,
                 sem_g, sem_o, sem_p, sem_k):
    wid = lax.axis_index("s") * NC + lax.axis_index("c")
    base = wid * BPW

    pltpu.sync_copy(idx_hbm, idx_v)

    s_slot = wid // 4
    s_row = wid % 4
    cp_p = pltpu.async_copy(pose_hbm.at[s_slot, s_row], pose_v, sem_p)
    cp_k = pltpu.async_copy(mask_hbm.at[pl.ds(0, KHALF)], mask_tab_v, sem_k)

    lane = lax.iota(jnp.int32, 16)

    def pose_range(lo, n):
        def body(i, _):
            for u in range(UNROLL):
                g = lo + i * UNROLL + u
                obj = idx_v[pl.ds(g * 16, 16)]
                pos = g * 16 + lane
                for b in range(4):
                    bvec = jnp.full((16,), b, jnp.int32)
                    vals = plsc.load_gather(pose_v, [bvec, obj])
                    plsc.store_scatter(pout_v, [bvec, pos], vals)
            return 0
        lax.fori_loop(0, n // UNROLL, body, 0)

    quota = [0, 0, 0] + [20] * 12 + [16]
    q_lo = [sum(quota[:c]) for c in range(NCH)]

    def start_gather(c, buf):
        return pltpu.async_copy(
            mem_hbm.at[idx_v.at[pl.ds(base + c * CH, CH)]],
            mem_v.at[buf], sem_g)

    cps = [start_gather(0, 0)]
    outs = [None] * NCH
    for c in range(NCH):
        cps[c].wait()
        if c + 1 < NCH:
            if c >= 1:
                outs[c - 1].wait()
            cps.append(start_gather(c + 1, (c + 1) % 2))
        outs[c] = pltpu.async_copy(
            mem_v.at[c % 2], mem_out.at[pl.ds(base + c * CH, CH)], sem_o)
        if quota[c]:
            if q_lo[c] == 0:
                cp_p.wait()
            pose_range(q_lo[c], quota[c])
    outs[NCH - 2].wait()
    outs[NCH - 1].wait()
    pltpu.sync_copy(pout_v, pose_out.at[s_slot, s_row])

    for h in range(2):
        if h == 0:
            cp_k.wait()
        else:
            pltpu.async_copy(mask_hbm.at[pl.ds(KHALF, KHALF)],
                             mask_tab_v, sem_k).wait()
        for g in range(BPW // 16):
            obj = idx_v[pl.ds(base + g * 16, 16)]
            local = obj - h * KHALF
            ok = (local >= 0) & (local < KHALF)
            safe = jnp.clip(local, 0, KHALF - 1)
            vals = plsc.load_gather(mask_tab_v, [safe])
            plsc.store_scatter(mask_loc_v, [g * 16 + lane], vals, mask=ok)
    pltpu.sync_copy(mask_loc_v, mask_out.at[pl.ds(base, BPW)])


@jax.jit
def _retrieve(object_indices, memory, poses_t, mask_packed):
    mesh = plsc.VectorSubcoreMesh(core_axis_name="c", subcore_axis_name="s")
    run = functools.partial(
        pl.kernel,
        out_type=(
            jax.ShapeDtypeStruct((M, SLOTS, D_MEMORY), jnp.float32),
            jax.ShapeDtypeStruct((SLOTS, 4, 4, M), jnp.float32),
            jax.ShapeDtypeStruct((M,), jnp.int32),
        ),
        mesh=mesh,
        compiler_params=pltpu.CompilerParams(needs_layout_passes=False),
        scratch_types=[
            pltpu.VMEM((M,), jnp.int32),
            pltpu.VMEM((2, CH, SLOTS, D_MEMORY), jnp.float32),
            pltpu.VMEM((4, MAX_OBJECTS), jnp.float32),
            pltpu.VMEM((4, M), jnp.float32),
            pltpu.VMEM((KHALF,), jnp.int32),
            pltpu.VMEM((BPW,), jnp.int32),
            pltpu.SemaphoreType.DMA,
            pltpu.SemaphoreType.DMA,
            pltpu.SemaphoreType.DMA,
            pltpu.SemaphoreType.DMA,
        ],
    )(_gather_body)
    return run(object_indices, memory, poses_t, mask_packed)


def kernel(object_indices, memory, capture_poses, slot_filled):
    idx = object_indices.astype(jnp.int32)
    poses_t = jnp.transpose(capture_poses, (1, 2, 3, 0))
    mask_packed = jnp.sum(
        slot_filled.astype(jnp.int32) << jnp.arange(SLOTS, dtype=jnp.int32),
        axis=1, dtype=jnp.int32)

    memory_entries, pose_o, mask_o = _retrieve(idx, memory, poses_t,
                                               mask_packed)

    entry_poses = jnp.transpose(pose_o, (3, 0, 1, 2))
    entry_mask = (
        (mask_o[:, None] >> jnp.arange(SLOTS, dtype=jnp.int32)) & 1) != 0
    return (memory_entries, entry_poses, entry_mask)


# speedup vs baseline: 1.0070x; 1.0070x over previous
import functools

import jax
import jax.numpy as jnp
from jax import lax
from jax.experimental import pallas as pl
from jax.experimental.pallas import tpu as pltpu
from jax.experimental.pallas import tpu_sc as plsc

MAX_OBJECTS = 16384
SLOTS = 8
D_MEMORY = 256
M = 4096

NC = 2
NS = 16
NW = NC * NS
BPW = M // NW
CH = 8
NCH = BPW // CH
KHALF = MAX_OBJECTS // 2
NGRP = M // 16
UNROLL = 4


def _gather_body(idx_hbm, mem_hbm, pose_hbm, mask_hbm,
                 mem_out, pose_out, mask_out,
                 idx_v, mem_v, pose_v, pout_v, mask_tab_v, mask_loc_v,
                 sem_g, sem_o, sem_p, sem_k):
    wid = lax.axis_index("s") * NC + lax.axis_index("c")
    base = wid * BPW

    pltpu.sync_copy(idx_hbm, idx_v)

    s_slot = wid // 4
    s_row = wid % 4

    lane = lax.iota(jnp.int32, 16)

    def pose_range(lo, n):
        def body(i, _):
            for u in range(UNROLL):
                g = lo + i * UNROLL + u
                obj = idx_v[pl.ds(g * 16, 16)]
                pos = g * 16 + lane
                for b in range(4):
                    bvec = jnp.full((16,), b, jnp.int32)
                    vals = plsc.load_gather(pose_v, [bvec, obj])
                    plsc.store_scatter(pout_v, [bvec, pos], vals)
            return 0
        lax.fori_loop(0, n // UNROLL, body, 0)

    quota = [0] * 4 + [24] * 8 + [16] * 4
    q_lo = [sum(quota[:c]) for c in range(NCH)]

    def start_gather(c, buf):
        return pltpu.async_copy(
            mem_hbm.at[idx_v.at[pl.ds(base + c * CH, CH)]],
            mem_v.at[buf], sem_g)

    cps = [start_gather(0, 0), start_gather(1, 1)]
    cp_k = pltpu.async_copy(mask_hbm.at[pl.ds(0, KHALF)], mask_tab_v, sem_k)
    cp_p = pltpu.async_copy(pose_hbm.at[s_slot, s_row], pose_v, sem_p)
    outs = [None] * NCH
    for c in range(NCH):
        cps[c].wait()
        if 1 <= c < NCH - 1:
            outs[c - 1].wait()
            cps.append(start_gather(c + 1, (c + 1) % 2))
        outs[c] = pltpu.async_copy(
            mem_v.at[c % 2], mem_out.at[pl.ds(base + c * CH, CH)], sem_o)
        if quota[c]:
            if q_lo[c] == 0:
                cp_p.wait()
            pose_range(q_lo[c], quota[c])
    outs[NCH - 2].wait()
    outs[NCH - 1].wait()
    pltpu.sync_copy(pout_v, pose_out.at[s_slot, s_row])

    for h in range(2):
        if h == 0:
            cp_k.wait()
        else:
            pltpu.async_copy(mask_hbm.at[pl.ds(KHALF, KHALF)],
                             mask_tab_v, sem_k).wait()
        for g in range(BPW // 16):
            obj = idx_v[pl.ds(base + g * 16, 16)]
            local = obj - h * KHALF
            ok = (local >= 0) & (local < KHALF)
            safe = jnp.clip(local, 0, KHALF - 1)
            vals = plsc.load_gather(mask_tab_v, [safe])
            plsc.store_scatter(mask_loc_v, [g * 16 + lane], vals, mask=ok)
    pltpu.sync_copy(mask_loc_v, mask_out.at[pl.ds(base, BPW)])


@jax.jit
def _retrieve(object_indices, memory, poses_t, mask_packed):
    mesh = plsc.VectorSubcoreMesh(core_axis_name="c", subcore_axis_name="s")
    run = functools.partial(
        pl.kernel,
        out_type=(
            jax.ShapeDtypeStruct((M, SLOTS, D_MEMORY), jnp.float32),
            jax.ShapeDtypeStruct((SLOTS, 4, 4, M), jnp.float32),
            jax.ShapeDtypeStruct((M,), jnp.int32),
        ),
        mesh=mesh,
        compiler_params=pltpu.CompilerParams(needs_layout_passes=False),
        scratch_types=[
            pltpu.VMEM((M,), jnp.int32),
            pltpu.VMEM((2, CH, SLOTS, D_MEMORY), jnp.float32),
            pltpu.VMEM((4, MAX_OBJECTS), jnp.float32),
            pltpu.VMEM((4, M), jnp.float32),
            pltpu.VMEM((KHALF,), jnp.int32),
            pltpu.VMEM((BPW,), jnp.int32),
            pltpu.SemaphoreType.DMA,
            pltpu.SemaphoreType.DMA,
            pltpu.SemaphoreType.DMA,
            pltpu.SemaphoreType.DMA,
        ],
    )(_gather_body)
    return run(object_indices, memory, poses_t, mask_packed)


def kernel(object_indices, memory, capture_poses, slot_filled):
    idx = object_indices.astype(jnp.int32)
    poses_t = jnp.transpose(capture_poses, (1, 2, 3, 0))
    mask_packed = jnp.sum(
        slot_filled.astype(jnp.int32) << jnp.arange(SLOTS, dtype=jnp.int32),
        axis=1, dtype=jnp.int32)

    memory_entries, pose_o, mask_o = _retrieve(idx, memory, poses_t,
                                               mask_packed)

    entry_poses = jnp.transpose(pose_o, (3, 0, 1, 2))
    entry_mask = (
        (mask_o[:, None] >> jnp.arange(SLOTS, dtype=jnp.int32)) & 1) != 0
    return (memory_entries, entry_poses, entry_mask)


# speedup vs baseline: 1.0640x; 1.0566x over previous
import functools

import jax
import jax.numpy as jnp
from jax import lax
from jax.experimental import pallas as pl
from jax.experimental.pallas import tpu as pltpu
from jax.experimental.pallas import tpu_sc as plsc

MAX_OBJECTS = 16384
SLOTS = 8
D_MEMORY = 256
M = 4096

NC = 2
NS = 16
NW = NC * NS
BPW = M // NW
CH = 8
NB = 4
PH2 = MAX_OBJECTS // 2
NCH = BPW // CH
KHALF = MAX_OBJECTS // 2
NGRP = M // 16
UNROLL = 4


def _gather_body(idx_hbm, mem_hbm, pose_hbm, mask_hbm,
                 mem_out, pose_out, mask_out,
                 idx_v, mem_v, pose_v, pout_v, mask_tab_v, mask_loc_v,
                 sem_g, sem_o, sem_p, sem_k):
    wid = lax.axis_index("s") * NC + lax.axis_index("c")
    base = wid * BPW

    pltpu.sync_copy(idx_hbm, idx_v)

    s_slot = wid // 4
    s_row = wid % 4

    lane = lax.iota(jnp.int32, 16)

    def pose_range(h, lo, n):
        def body(i, _):
            for u in range(UNROLL):
                g = lo + i * UNROLL + u
                obj = idx_v[pl.ds(g * 16, 16)]
                local = obj - h * PH2
                ok = (local >= 0) & (local < PH2)
                safe = jnp.clip(local, 0, PH2 - 1)
                pos = g * 16 + lane
                for b in range(4):
                    bvec = jnp.full((16,), b, jnp.int32)
                    vals = plsc.load_gather(pose_v, [bvec, safe])
                    plsc.store_scatter(pout_v, [bvec, pos], vals, mask=ok)
            return 0
        lax.fori_loop(0, n // UNROLL, body, 0)

    quota0 = [0, 0, 0] + [44] * 4 + [40, 40] + [0] * 7
    quota1 = [0] * 11 + [52] * 4 + [48]
    q0_lo = [sum(quota0[:c]) for c in range(NCH)]
    q1_lo = [sum(quota1[:c]) for c in range(NCH)]

    def start_gather(c, buf):
        return pltpu.async_copy(
            mem_hbm.at[idx_v.at[pl.ds(base + c * CH, CH)]],
            mem_v.at[buf], sem_g)

    cps = [start_gather(0, 0), start_gather(1, 1)]
    cp_k = pltpu.async_copy(mask_hbm.at[pl.ds(0, KHALF)], mask_tab_v, sem_k)
    cp_p = pltpu.async_copy(
        pose_hbm.at[s_slot, s_row, :, pl.ds(0, PH2)], pose_v, sem_p)
    outs = [None] * NCH
    for c in range(NCH):
        cps[c].wait()
        outs[c] = pltpu.async_copy(
            mem_v.at[c % NB], mem_out.at[pl.ds(base + c * CH, CH)], sem_o)
        g = c + 2
        if g < NCH:
            if g >= NB:
                outs[g - NB].wait()
            cps.append(start_gather(g, g % NB))
        if quota0[c]:
            if q0_lo[c] == 0:
                cp_p.wait()
            pose_range(0, q0_lo[c], quota0[c])
        if c == 8:
            cp_p = pltpu.async_copy(
                pose_hbm.at[s_slot, s_row, :, pl.ds(PH2, PH2)], pose_v, sem_p)
        if quota1[c]:
            if q1_lo[c] == 0:
                cp_p.wait()
            pose_range(1, q1_lo[c], quota1[c])
    for c in range(NCH - NB, NCH):
        outs[c].wait()
    pltpu.sync_copy(pout_v, pose_out.at[s_slot, s_row])

    for h in range(2):
        if h == 0:
            cp_k.wait()
        else:
            pltpu.async_copy(mask_hbm.at[pl.ds(KHALF, KHALF)],
                             mask_tab_v, sem_k).wait()
        for g in range(BPW // 16):
            obj = idx_v[pl.ds(base + g * 16, 16)]
            local = obj - h * KHALF
            ok = (local >= 0) & (local < KHALF)
            safe = jnp.clip(local, 0, KHALF - 1)
            vals = plsc.load_gather(mask_tab_v, [safe])
            plsc.store_scatter(mask_loc_v, [g * 16 + lane], vals, mask=ok)
    pltpu.sync_copy(mask_loc_v, mask_out.at[pl.ds(base, BPW)])


@jax.jit
def _retrieve(object_indices, memory, poses_t, mask_packed):
    mesh = plsc.VectorSubcoreMesh(core_axis_name="c", subcore_axis_name="s")
    run = functools.partial(
        pl.kernel,
        out_type=(
            jax.ShapeDtypeStruct((M, SLOTS, D_MEMORY), jnp.float32),
            jax.ShapeDtypeStruct((SLOTS, 4, 4, M), jnp.float32),
            jax.ShapeDtypeStruct((M,), jnp.int32),
        ),
        mesh=mesh,
        compiler_params=pltpu.CompilerParams(needs_layout_passes=False),
        scratch_types=[
            pltpu.VMEM((M,), jnp.int32),
            pltpu.VMEM((NB, CH, SLOTS, D_MEMORY), jnp.float32),
            pltpu.VMEM((4, PH2), jnp.float32),
            pltpu.VMEM((4, M), jnp.float32),
            pltpu.VMEM((KHALF,), jnp.int32),
            pltpu.VMEM((BPW,), jnp.int32),
            pltpu.SemaphoreType.DMA,
            pltpu.SemaphoreType.DMA,
            pltpu.SemaphoreType.DMA,
            pltpu.SemaphoreType.DMA,
        ],
    )(_gather_body)
    return run(object_indices, memory, poses_t, mask_packed)


def kernel(object_indices, memory, capture_poses, slot_filled):
    idx = object_indices.astype(jnp.int32)
    poses_t = jnp.transpose(capture_poses, (1, 2, 3, 0))
    mask_packed = jnp.sum(
        slot_filled.astype(jnp.int32) << jnp.arange(SLOTS, dtype=jnp.int32),
        axis=1, dtype=jnp.int32)

    memory_entries, pose_o, mask_o = _retrieve(idx, memory, poses_t,
                                               mask_packed)

    entry_poses = jnp.transpose(pose_o, (3, 0, 1, 2))
    entry_mask = (
        (mask_o[:, None] >> jnp.arange(SLOTS, dtype=jnp.int32)) & 1) != 0
    return (memory_entries, entry_poses, entry_mask)
